# baseline (speedup 1.0000x reference)
import math

import jax
import jax.numpy as jnp
import numpy as np
from jax import lax
from jax.experimental import pallas as pl
from jax.experimental.pallas import tpu as pltpu
from jax.experimental.pallas import tpu_sc as plsc

VOCAB = 1000000
D = 64
B = 1024
L = 200

_NC = 2
_NS = 16
_NW = _NC * _NS
_N = B * L
_PER_W = _N // _NW
_SUB = 128
_NSUB = _PER_W // _SUB


def _pos_encoding_np(max_len, d):
    pos = np.arange(max_len)[:, None].astype(np.float32)
    i = np.arange(0, d, 2).astype(np.float32)
    div = np.exp(-math.log(10000.0) * i / float(d))
    pe = np.zeros((max_len, d), dtype=np.float32)
    pe[:, 0::2] = np.sin(pos * div)
    pe[:, 1::2] = np.cos(pos * div)
    return pe


_PE_FLAT = _pos_encoding_np(L, D).reshape(-1)


def _emb_body(idx_hbm, pe_hbm, table_hbm, out_hbm,
              idx_v, lid_v, pe_v, gb0, gb1, st0, st1,
              gsem0, gsem1, ssem0, ssem1):
    wid = lax.axis_index("s") * _NC + lax.axis_index("c")
    base = wid * _PER_W
    iota = lax.broadcasted_iota(jnp.int32, (16,), 0)
    pltpu.sync_copy(idx_hbm.at[pl.ds(base, _PER_W)], idx_v)
    pltpu.sync_copy(pe_hbm, pe_v)

    @pl.loop(0, _PER_W // 16, unroll=8)
    def _mklid(g):
        lid_v[pl.ds(g * 16, 16)] = idx_v[pl.ds(g * 16, 16)] >> 1

    bufs = ((gb0, st0, gsem0, ssem0), (gb1, st1, gsem1, ssem1))

    def start_gather(g, gb, gsem):
        pltpu.async_copy(table_hbm.at[lid_v.at[pl.ds(g * _SUB, _SUB)]],
                         gb, gsem)

    def wait_gather(gb, gsem):
        pltpu.make_async_copy(table_hbm.at[pl.ds(0, _SUB)], gb, gsem).wait()

    def start_store(g, st, ssem):
        pltpu.async_copy(st, out_hbm.at[pl.ds(base + g * _SUB, _SUB)], ssem)

    def wait_store(st, ssem):
        pltpu.make_async_copy(table_hbm.at[pl.ds(0, _SUB)], st, ssem).wait()

    def extract(g, gb, st):
        for j0 in range(0, _SUB, 16):
            xv = idx_v[pl.ds(g * _SUB + j0, 16)]
            par64 = (xv & 1) << 6
            nv = base + g * _SUB + j0 + iota
            pos64 = lax.rem(nv, L) * D
            rows = iota + j0

            @pl.loop(0, D, unroll=8)
            def _feat(c):
                tok = plsc.load_gather(gb, [rows, par64 + c])
                pv = plsc.load_gather(pe_v, [pos64 + c])
                plsc.store_scatter(st, [rows, iota * 0 + c], tok + pv)

    start_gather(0, gb0, gsem0)

    @pl.loop(0, _NSUB // 2)
    def _step(h):
        for b in range(2):
            gb, st, gsem, ssem = bufs[b]
            ogb, ost, ogsem, ossem = bufs[1 - b]
            g = h * 2 + b
            wait_gather(gb, gsem)
            if b == 0:
                @pl.when(h >= 1)
                def _w():
                    wait_store(ost, ossem)

                start_gather(g + 1, ogb, ogsem)
            else:
                wait_store(ost, ossem)

                @pl.when(h < _NSUB // 2 - 1)
                def _g():
                    start_gather(g + 1, ogb, ogsem)

            extract(g, gb, st)
            start_store(g, st, ssem)

    wait_store(st1, ssem1)


@jax.jit
def _emb(xi, pe, t2):
    f = pl.kernel(
        _emb_body,
        mesh=plsc.VectorSubcoreMesh(core_axis_name="c", subcore_axis_name="s"),
        out_type=jax.ShapeDtypeStruct((_N, 128), jnp.float32),
        scratch_types=[
            pltpu.VMEM((_PER_W,), jnp.int32),
            pltpu.VMEM((_PER_W,), jnp.int32),
            pltpu.VMEM((L * D,), jnp.float32),
            pltpu.VMEM((_SUB, 128), jnp.float32),
            pltpu.VMEM((_SUB, 128), jnp.float32),
            pltpu.VMEM((_SUB, 128), jnp.float32),
            pltpu.VMEM((_SUB, 128), jnp.float32),
            pltpu.SemaphoreType.DMA,
            pltpu.SemaphoreType.DMA,
            pltpu.SemaphoreType.DMA,
            pltpu.SemaphoreType.DMA,
        ],
        compiler_params=pltpu.CompilerParams(
            use_tc_tiling_on_sc=True, needs_layout_passes=False
        ),
    )
    return f(xi, pe, t2)


def kernel(x, table):
    t2 = table.reshape(VOCAB // 2, 128)
    xi = x.astype(jnp.int32).reshape(_N)
    padded = _emb(xi, jnp.asarray(_PE_FLAT), t2)
    return padded[:, :64].reshape(B, L, D)


# speedup vs baseline: 1.6529x; 1.6529x over previous
import math

import jax
import jax.numpy as jnp
import numpy as np
from jax import lax
from jax.experimental import pallas as pl
from jax.experimental.pallas import tpu as pltpu
from jax.experimental.pallas import tpu_sc as plsc

D = 64
B = 1024
L = 200

_NC = 2
_NS = 16
_NW = _NC * _NS
_N = B * L
_PER_W = _N // _NW
_SUB = 128
_NSUB = _PER_W // _SUB


def _pos_encoding_np(max_len, d):
    pos = np.arange(max_len)[:, None].astype(np.float32)
    i = np.arange(0, d, 2).astype(np.float32)
    div = np.exp(-math.log(10000.0) * i / float(d))
    pe = np.zeros((max_len, d), dtype=np.float32)
    pe[:, 0::2] = np.sin(pos * div)
    pe[:, 1::2] = np.cos(pos * div)
    return pe


_PE2 = np.tile(_pos_encoding_np(L, D), (2, 1))


def _emb_body(idx_hbm, pe_hbm, table_hbm, out_hbm,
              idx_v, pe_v, row0, row1, gsem0, gsem1, ssem0, ssem1):
    wid = lax.axis_index("s") * _NC + lax.axis_index("c")
    base = wid * _PER_W
    pltpu.sync_copy(idx_hbm.at[pl.ds(base, _PER_W)], idx_v)
    pltpu.sync_copy(pe_hbm, pe_v)

    bufs = ((row0, gsem0, ssem0), (row1, gsem1, ssem1))

    def start_gather(g, row, gsem):
        pltpu.async_copy(table_hbm.at[idx_v.at[pl.ds(g * _SUB, _SUB)]], row, gsem)

    def wait_gather(row, gsem):
        pltpu.make_async_copy(table_hbm.at[pl.ds(0, _SUB)], row, gsem).wait()

    def start_store(g, row, ssem):
        pltpu.async_copy(row, out_hbm.at[pl.ds(base + g * _SUB, _SUB)], ssem)

    def wait_store(row, ssem):
        pltpu.make_async_copy(table_hbm.at[pl.ds(0, _SUB)], row, ssem).wait()

    def add_pe(row, g):
        p0 = lax.rem(g * _SUB, L)

        @pl.loop(0, _SUB, unroll=8)
        def _add(j):
            pr = p0 + j
            for v in range(D // 16):
                sl = pl.ds(v * 16, 16)
                plsc.addupdate(row.at[j, sl], pe_v[pr, sl])

    start_gather(0, row0, gsem0)

    @pl.loop(0, _NSUB // 2)
    def _step(h):
        for b in range(2):
            row, gsem, ssem = bufs[b]
            orow, ogsem, ossem = bufs[1 - b]
            g = h * 2 + b
            wait_gather(row, gsem)
            if b == 0:
                @pl.when(h >= 1)
                def _w():
                    wait_store(orow, ossem)

                start_gather(g + 1, orow, ogsem)
            else:
                wait_store(orow, ossem)

                @pl.when(h < _NSUB // 2 - 1)
                def _g():
                    start_gather(g + 1, orow, ogsem)

            add_pe(row, g)
            start_store(g, row, ssem)

    wait_store(row1, ssem1)


@jax.jit
def _emb(xi, pe2, table):
    f = pl.kernel(
        _emb_body,
        mesh=plsc.VectorSubcoreMesh(core_axis_name="c", subcore_axis_name="s"),
        out_type=jax.ShapeDtypeStruct((_N, D), jnp.float32),
        scratch_types=[
            pltpu.VMEM((_PER_W,), jnp.int32),
            pltpu.VMEM((2 * L, D), jnp.float32),
            pltpu.VMEM((_SUB, D), jnp.float32),
            pltpu.VMEM((_SUB, D), jnp.float32),
            pltpu.SemaphoreType.DMA,
            pltpu.SemaphoreType.DMA,
            pltpu.SemaphoreType.DMA,
            pltpu.SemaphoreType.DMA,
        ],
        compiler_params=pltpu.CompilerParams(use_tc_tiling_on_sc=False),
    )
    return f(xi, pe2, table)


def kernel(x, table):
    xi = x.astype(jnp.int32).reshape(_N)
    out = _emb(xi, jnp.asarray(_PE2), table)
    return out.reshape(B, L, D)


# speedup vs baseline: 1.6581x; 1.0031x over previous
import math

import jax
import jax.numpy as jnp
import numpy as np
from jax import lax
from jax.experimental import pallas as pl
from jax.experimental.pallas import tpu as pltpu
from jax.experimental.pallas import tpu_sc as plsc

D = 64
B = 1024
L = 200

_NC = 2
_NS = 16
_NW = _NC * _NS
_N = B * L
_PER_W = _N // _NW
_SUB = 128
_NSUB = _PER_W // _SUB


def _pos_encoding_np(max_len, d):
    pos = np.arange(max_len)[:, None].astype(np.float32)
    i = np.arange(0, d, 2).astype(np.float32)
    div = np.exp(-math.log(10000.0) * i / float(d))
    pe = np.zeros((max_len, d), dtype=np.float32)
    pe[:, 0::2] = np.sin(pos * div)
    pe[:, 1::2] = np.cos(pos * div)
    return pe


_PE2 = np.tile(_pos_encoding_np(L, D), (2, 1))


def _emb_body(idx_hbm, pe_hbm, table_hbm, out_hbm,
              idx_v, pe_v, row0, row1, row2, row3,
              gsem0, gsem1, gsem2, gsem3, ssem0, ssem1, ssem2, ssem3):
    wid = lax.axis_index("s") * _NC + lax.axis_index("c")
    base = wid * _PER_W
    pltpu.sync_copy(idx_hbm.at[pl.ds(base, _PER_W)], idx_v)
    pltpu.sync_copy(pe_hbm, pe_v)

    bufs = ((row0, gsem0, ssem0), (row1, gsem1, ssem1),
            (row2, gsem2, ssem2), (row3, gsem3, ssem3))

    def start_gather(g, row, gsem):
        pltpu.async_copy(table_hbm.at[idx_v.at[pl.ds(g * _SUB, _SUB)]], row, gsem)

    def wait_gather(row, gsem):
        pltpu.make_async_copy(table_hbm.at[pl.ds(0, _SUB)], row, gsem).wait()

    def start_store(g, row, ssem):
        pltpu.async_copy(row, out_hbm.at[pl.ds(base + g * _SUB, _SUB)], ssem)

    def wait_store(row, ssem):
        pltpu.make_async_copy(table_hbm.at[pl.ds(0, _SUB)], row, ssem).wait()

    def add_pe(row, g):
        p0 = lax.rem(g * _SUB, L)

        @pl.loop(0, _SUB, unroll=8)
        def _add(j):
            pr = p0 + j
            for v in range(D // 16):
                sl = pl.ds(v * 16, 16)
                plsc.addupdate(row.at[j, sl], pe_v[pr, sl])

    for p in range(3):
        start_gather(p, bufs[p][0], bufs[p][1])

    @pl.loop(0, (_NSUB - 2) // 4)
    def _step(q):
        for b4 in range(4):
            row, gsem, ssem = bufs[b4]
            nrow, ngsem, nssem = bufs[(b4 + 3) % 4]
            g = q * 4 + b4
            wait_gather(row, gsem)

            @pl.when(g + 3 < _NSUB)
            def _pref():
                @pl.when(g >= 1)
                def _w():
                    wait_store(nrow, nssem)

                start_gather(g + 3, nrow, ngsem)

            add_pe(row, g)
            start_store(g, row, ssem)

    for g in (_NSUB - 2, _NSUB - 1):
        row, gsem, ssem = bufs[g % 4]
        wait_gather(row, gsem)
        add_pe(row, g)
        start_store(g, row, ssem)
    for g in range(_NSUB - 4, _NSUB):
        row, gsem, ssem = bufs[g % 4]
        wait_store(row, ssem)


@jax.jit
def _emb(xi, pe2, table):
    f = pl.kernel(
        _emb_body,
        mesh=plsc.VectorSubcoreMesh(core_axis_name="c", subcore_axis_name="s"),
        out_type=jax.ShapeDtypeStruct((_N, D), jnp.float32),
        scratch_types=[
            pltpu.VMEM((_PER_W,), jnp.int32),
            pltpu.VMEM((2 * L, D), jnp.float32),
            pltpu.VMEM((_SUB, D), jnp.float32),
            pltpu.VMEM((_SUB, D), jnp.float32),
            pltpu.VMEM((_SUB, D), jnp.float32),
            pltpu.VMEM((_SUB, D), jnp.float32),
            pltpu.SemaphoreType.DMA,
            pltpu.SemaphoreType.DMA,
            pltpu.SemaphoreType.DMA,
            pltpu.SemaphoreType.DMA,
            pltpu.SemaphoreType.DMA,
            pltpu.SemaphoreType.DMA,
            pltpu.SemaphoreType.DMA,
            pltpu.SemaphoreType.DMA,
        ],
        compiler_params=pltpu.CompilerParams(use_tc_tiling_on_sc=False),
    )
    return f(xi, pe2, table)


def kernel(x, table):
    xi = x.astype(jnp.int32).reshape(_N)
    out = _emb(xi, jnp.asarray(_PE2), table)
    return out.reshape(B, L, D)
